# baseline (speedup 1.0000x reference)
import functools

import jax
import jax.numpy as jnp
from jax.experimental import pallas as pl
from jax.experimental.pallas import tpu as pltpu

_EB = 512
_EB_DEG = 256


def _deg_cast_body(n_eb, E, hv_ref, hb_ref, dv_ref, de_ref):
    e = pl.program_id(0)
    eb = _EB_DEG
    hw = hv_ref[...]
    lane = jax.lax.broadcasted_iota(jnp.int32, (1, 2 * eb), 1)
    keep = ((lane % 2) == 1) & ((lane + e * 2 * eb) < 2 * E)
    hcl = jnp.where(keep, hw, jnp.bfloat16(0.0))
    row = jax.lax.broadcasted_iota(jnp.int32, (2 * eb, eb), 0)
    col = jax.lax.broadcasted_iota(jnp.int32, (2 * eb, eb), 1)
    sel = (row == 2 * col + 1).astype(jnp.bfloat16)
    hb = jax.lax.dot_general(hcl, sel, (((1,), (0,)), ((), ())),
                             preferred_element_type=jnp.float32)
    hb_ref[...] = hb.astype(jnp.bfloat16)
    cs = jnp.sum(hb, axis=0)
    de_ref[...] = (1.0 / jnp.maximum(cs, 1e-12)).reshape(de_ref.shape)
    rs = jnp.sum(hb, axis=1, keepdims=True)

    @pl.when(e == 0)
    def _():
        dv_ref[...] = rs

    @pl.when(e != 0)
    def _():
        dv_ref[...] = dv_ref[...] + rs

    @pl.when(e == n_eb - 1)
    def _():
        dv_ref[...] = 1.0 / jnp.sqrt(jnp.maximum(dv_ref[...], 1e-12))


def _layer_body(n_eb, residual, h_ref, hb_ref, de_ref, dv_ref, w_ref, b_ref,
                o_ref, tT_scr, acc_scr):
    e = pl.program_id(0)

    @pl.when(e == 0)
    def _():
        t = (h_ref[...] * dv_ref[...]).astype(jnp.bfloat16)
        tT_scr[...] = t.T
        acc_scr[...] = jnp.zeros_like(acc_scr)

    hb = hb_ref[...]
    sT = jax.lax.dot_general(tT_scr[...], hb, (((1,), (0,)), ((), ())),
                             preferred_element_type=jnp.float32)
    sT = sT * de_ref[0]
    s = sT.astype(jnp.bfloat16).T
    acc_scr[...] += jax.lax.dot_general(hb, s, (((1,), (0,)), ((), ())),
                                        preferred_element_type=jnp.float32)

    @pl.when(e == n_eb - 1)
    def _():
        g = acc_scr[...] * dv_ref[...]
        o = jax.lax.dot_general(g, w_ref[...], (((1,), (0,)), ((), ())),
                                preferred_element_type=jnp.float32)
        o = jnp.maximum(o + b_ref[...], 0.0)
        if residual:
            o = o + h_ref[...]
        o_ref[...] = o


def kernel(x, H, W0, b0, W1, b1, W2, b2):
    N, _ = x.shape
    E = H.shape[1]
    n_eb = -(-E // _EB)
    E_pad = n_eb * _EB
    n_deg = E_pad // _EB_DEG

    Hv = jax.lax.bitcast_convert_type(H, jnp.uint16).reshape(N, 2 * E)
    Hv = jax.lax.bitcast_convert_type(Hv, jnp.bfloat16)

    hb, dv, de3 = pl.pallas_call(
        functools.partial(_deg_cast_body, n_deg, E),
        grid=(n_deg,),
        in_specs=[pl.BlockSpec((N, 2 * _EB_DEG), lambda e: (0, e))],
        out_specs=[
            pl.BlockSpec((N, _EB_DEG), lambda e: (0, e)),
            pl.BlockSpec((N, 1), lambda e: (0, 0)),
            pl.BlockSpec((1, 1, _EB_DEG), lambda e: (e, 0, 0)),
        ],
        out_shape=[
            jax.ShapeDtypeStruct((N, E_pad), jnp.bfloat16),
            jax.ShapeDtypeStruct((N, 1), jnp.float32),
            jax.ShapeDtypeStruct((n_deg, 1, _EB_DEG), jnp.float32),
        ],
    )(Hv)
    de = de3.reshape(n_eb, 1, _EB)

    def layer(h, w, b, residual):
        d = h.shape[1]
        dout = w.shape[1]
        return pl.pallas_call(
            functools.partial(_layer_body, n_eb, residual),
            grid=(n_eb,),
            in_specs=[
                pl.BlockSpec((N, d), lambda e: (0, 0)),
                pl.BlockSpec((N, _EB), lambda e: (0, e)),
                pl.BlockSpec((1, 1, _EB), lambda e: (e, 0, 0)),
                pl.BlockSpec((N, 1), lambda e: (0, 0)),
                pl.BlockSpec((d, dout), lambda e: (0, 0)),
                pl.BlockSpec((1, dout), lambda e: (0, 0)),
            ],
            out_specs=pl.BlockSpec((N, dout), lambda e: (0, 0)),
            out_shape=jax.ShapeDtypeStruct((N, dout), jnp.float32),
            scratch_shapes=[
                pltpu.VMEM((d, N), jnp.bfloat16),
                pltpu.VMEM((N, d), jnp.float32),
            ],
        )(h, hb, de, dv, w, b)

    h0 = layer(x, W0, b0.reshape(1, -1), residual=False)
    h1 = layer(h0, W1, b1.reshape(1, -1), residual=True)
    h2 = layer(h1, W2, b2.reshape(1, -1), residual=False)
    return h2


# speedup vs baseline: 7.8679x; 7.8679x over previous
import functools

import jax
import jax.numpy as jnp
from jax.experimental import pallas as pl
from jax.experimental.pallas import tpu as pltpu

_EB = 640
_EB_DEG = 256


def _deg_cast_body(n_eb, E, h_ref, hb_ref, dv_ref, de_ref):
    e = pl.program_id(0)
    eb = h_ref.shape[1]
    valid = (jax.lax.broadcasted_iota(jnp.int32, (1, eb), 1) + e * eb) < E
    h = jnp.where(valid, h_ref[...], 0.0)
    hb_ref[...] = h.astype(jnp.bfloat16)
    cs = jnp.sum(h, axis=0)
    de_ref[...] = (1.0 / jnp.maximum(cs, 1e-12)).reshape(de_ref.shape)
    rs = jnp.sum(h, axis=1, keepdims=True)

    @pl.when(e == 0)
    def _():
        dv_ref[...] = rs

    @pl.when(e != 0)
    def _():
        dv_ref[...] = dv_ref[...] + rs

    @pl.when(e == n_eb - 1)
    def _():
        dv_ref[...] = 1.0 / jnp.sqrt(jnp.maximum(dv_ref[...], 1e-12))


def _layer_body(n_eb, residual, h_ref, hb_ref, de_ref, dv_ref, w_ref, b_ref,
                o_ref, tT_scr, acc_scr):
    e = pl.program_id(0)

    @pl.when(e == 0)
    def _():
        t = (h_ref[...] * dv_ref[...]).astype(jnp.bfloat16)
        tT_scr[...] = t.T
        acc_scr[...] = jnp.zeros_like(acc_scr)

    hb = hb_ref[...]
    sT = jax.lax.dot_general(tT_scr[...], hb, (((1,), (0,)), ((), ())),
                             preferred_element_type=jnp.float32)
    sT = sT * de_ref[0]
    s = sT.astype(jnp.bfloat16).T
    acc_scr[...] += jax.lax.dot_general(hb, s, (((1,), (0,)), ((), ())),
                                        preferred_element_type=jnp.float32)

    @pl.when(e == n_eb - 1)
    def _():
        g = acc_scr[...] * dv_ref[...]
        o = jax.lax.dot_general(g, w_ref[...], (((1,), (0,)), ((), ())),
                                preferred_element_type=jnp.float32)
        o = jnp.maximum(o + b_ref[...], 0.0)
        if residual:
            o = o + h_ref[...]
        o_ref[...] = o


def kernel(x, H, W0, b0, W1, b1, W2, b2):
    N, _ = x.shape
    E = H.shape[1]
    n_eb = -(-E // _EB)
    E_pad = n_eb * _EB
    n_deg = E_pad // _EB_DEG

    hb, dv, de3 = pl.pallas_call(
        functools.partial(_deg_cast_body, n_deg, E),
        grid=(n_deg,),
        in_specs=[pl.BlockSpec((N, _EB_DEG), lambda e: (0, e))],
        out_specs=[
            pl.BlockSpec((N, _EB_DEG), lambda e: (0, e)),
            pl.BlockSpec((N, 1), lambda e: (0, 0)),
            pl.BlockSpec((1, 1, _EB_DEG), lambda e: (e, 0, 0)),
        ],
        out_shape=[
            jax.ShapeDtypeStruct((N, E_pad), jnp.bfloat16),
            jax.ShapeDtypeStruct((N, 1), jnp.float32),
            jax.ShapeDtypeStruct((n_deg, 1, _EB_DEG), jnp.float32),
        ],
    )(H)
    de = de3.reshape(n_eb, 1, _EB)

    def layer(h, w, b, residual):
        d = h.shape[1]
        dout = w.shape[1]
        return pl.pallas_call(
            functools.partial(_layer_body, n_eb, residual),
            grid=(n_eb,),
            in_specs=[
                pl.BlockSpec((N, d), lambda e: (0, 0)),
                pl.BlockSpec((N, _EB), lambda e: (0, e)),
                pl.BlockSpec((1, 1, _EB), lambda e: (e, 0, 0)),
                pl.BlockSpec((N, 1), lambda e: (0, 0)),
                pl.BlockSpec((d, dout), lambda e: (0, 0)),
                pl.BlockSpec((1, dout), lambda e: (0, 0)),
            ],
            out_specs=pl.BlockSpec((N, dout), lambda e: (0, 0)),
            out_shape=jax.ShapeDtypeStruct((N, dout), jnp.float32),
            scratch_shapes=[
                pltpu.VMEM((d, N), jnp.bfloat16),
                pltpu.VMEM((N, d), jnp.float32),
            ],
        )(h, hb, de, dv, w, b)

    h0 = layer(x, W0, b0.reshape(1, -1), residual=False)
    h1 = layer(h0, W1, b1.reshape(1, -1), residual=True)
    h2 = layer(h1, W2, b2.reshape(1, -1), residual=False)
    return h2


# speedup vs baseline: 8.2367x; 1.0469x over previous
import functools

import jax
import jax.numpy as jnp
from jax.experimental import pallas as pl
from jax.experimental.pallas import tpu as pltpu

_EB = 512
_EB_DEG = 256


def _deg_cast_body(n_eb, E, h_ref, hb_ref, dv_ref, de_ref):
    e = pl.program_id(0)
    eb = h_ref.shape[1]
    valid = (jax.lax.broadcasted_iota(jnp.int32, (1, eb), 1) + e * eb) < E
    h = jnp.where(valid, h_ref[...], 0.0)
    hb_ref[...] = h.astype(jnp.bfloat16)
    cs = jnp.sum(h, axis=0)
    de_ref[...] = (1.0 / jnp.maximum(cs, 1e-12)).reshape(de_ref.shape)
    rs = jnp.sum(h, axis=1, keepdims=True)

    @pl.when(e == 0)
    def _():
        dv_ref[...] = rs

    @pl.when(e != 0)
    def _():
        dv_ref[...] = dv_ref[...] + rs

    @pl.when(e == n_eb - 1)
    def _():
        dv_ref[...] = 1.0 / jnp.sqrt(jnp.maximum(dv_ref[...], 1e-12))


def _layer_body(n_eb, residual, h_ref, hb_ref, de_ref, dv_ref, w_ref, b_ref,
                o_ref, tT_scr, acc_scr):
    e = pl.program_id(0)

    @pl.when(e == 0)
    def _():
        t = (h_ref[...] * dv_ref[...]).astype(jnp.bfloat16)
        tT_scr[...] = t.T
        acc_scr[...] = jnp.zeros_like(acc_scr)

    hb = hb_ref[...]
    sT = jax.lax.dot_general(tT_scr[...], hb, (((1,), (0,)), ((), ())),
                             preferred_element_type=jnp.float32)
    sT = sT * de_ref[0]
    s = sT.astype(jnp.bfloat16).T
    acc_scr[...] += jax.lax.dot_general(hb, s, (((1,), (0,)), ((), ())),
                                        preferred_element_type=jnp.float32)

    @pl.when(e == n_eb - 1)
    def _():
        g = acc_scr[...] * dv_ref[...]
        o = jax.lax.dot_general(g, w_ref[...], (((1,), (0,)), ((), ())),
                                preferred_element_type=jnp.float32)
        o = jnp.maximum(o + b_ref[...], 0.0)
        if residual:
            o = o + h_ref[...]
        o_ref[...] = o


def kernel(x, H, W0, b0, W1, b1, W2, b2):
    N, _ = x.shape
    E = H.shape[1]
    n_eb = -(-E // _EB)
    E_pad = n_eb * _EB
    n_deg = E_pad // _EB_DEG

    hb, dv, de3 = pl.pallas_call(
        functools.partial(_deg_cast_body, n_deg, E),
        grid=(n_deg,),
        in_specs=[pl.BlockSpec((N, _EB_DEG), lambda e: (0, e))],
        out_specs=[
            pl.BlockSpec((N, _EB_DEG), lambda e: (0, e)),
            pl.BlockSpec((N, 1), lambda e: (0, 0)),
            pl.BlockSpec((1, 1, _EB_DEG), lambda e: (e, 0, 0)),
        ],
        out_shape=[
            jax.ShapeDtypeStruct((N, E_pad), jnp.bfloat16),
            jax.ShapeDtypeStruct((N, 1), jnp.float32),
            jax.ShapeDtypeStruct((n_deg, 1, _EB_DEG), jnp.float32),
        ],
    )(H)
    de = de3.reshape(n_eb, 1, _EB)

    def layer(h, w, b, residual):
        d = h.shape[1]
        dout = w.shape[1]
        return pl.pallas_call(
            functools.partial(_layer_body, n_eb, residual),
            grid=(n_eb,),
            in_specs=[
                pl.BlockSpec((N, d), lambda e: (0, 0)),
                pl.BlockSpec((N, _EB), lambda e: (0, e)),
                pl.BlockSpec((1, 1, _EB), lambda e: (e, 0, 0)),
                pl.BlockSpec((N, 1), lambda e: (0, 0)),
                pl.BlockSpec((d, dout), lambda e: (0, 0)),
                pl.BlockSpec((1, dout), lambda e: (0, 0)),
            ],
            out_specs=pl.BlockSpec((N, dout), lambda e: (0, 0)),
            out_shape=jax.ShapeDtypeStruct((N, dout), jnp.float32),
            scratch_shapes=[
                pltpu.VMEM((d, N), jnp.bfloat16),
                pltpu.VMEM((N, d), jnp.float32),
            ],
        )(h, hb, de, dv, w, b)

    h0 = layer(x, W0, b0.reshape(1, -1), residual=False)
    h1 = layer(h0, W1, b1.reshape(1, -1), residual=True)
    h2 = layer(h1, W2, b2.reshape(1, -1), residual=False)
    return h2
